# baseline (speedup 1.0000x reference)
import functools

import jax
import jax.numpy as jnp
from jax import lax
from jax.experimental import pallas as pl
from jax.experimental.pallas import tpu as pltpu

_THRESHOLD = 0.6
_SAFE_SPEED = 1
_SAFE_PRIORITY = 14
_SPEED_MAX = 8
_PRIORITY_MAX = 14
_LANES = 128


def _body(x_ref, w_ref, b_ref, o_ref, *, C):
    logits = jnp.dot(x_ref[...], w_ref[...],
                     preferred_element_type=jnp.float32) + b_ref[...]
    m = jnp.max(logits, axis=1, keepdims=True)
    e = jnp.exp(logits - m)
    s = jnp.sum(e, axis=1, keepdims=True)
    p = e / s
    ent = -jnp.sum(p * jnp.log(p + 1e-10), axis=1, keepdims=True)
    unc = ent / jnp.log(jnp.float32(C))

    pm = jnp.max(p, axis=1, keepdims=True)
    iota = lax.broadcasted_iota(jnp.int32, p.shape, 1)
    pred = jnp.min(jnp.where(p == pm, iota, _LANES), axis=1, keepdims=True)

    high = unc > _THRESHOLD
    is_speed = pred <= _SPEED_MAX
    is_prio = (pred > _SPEED_MAX) & (pred <= _PRIORITY_MAX)
    pred = jnp.where(high & is_speed, _SAFE_SPEED,
                     jnp.where(high & is_prio, _SAFE_PRIORITY, pred))

    out = jnp.where(iota == pred, jnp.float32(100.0), jnp.float32(-100.0))
    o_ref[...] = out[:, :C]


def kernel(x, W, b):
    N, D = x.shape
    C = W.shape[1]
    Wp = jnp.zeros((D, _LANES), W.dtype).at[:, :C].set(W)
    bp = jnp.full((1, _LANES), -1e30, jnp.float32).at[0, :C].set(b)
    BN = 4096
    return pl.pallas_call(
        functools.partial(_body, C=C),
        grid=(N // BN,),
        in_specs=[
            pl.BlockSpec((BN, D), lambda i: (i, 0)),
            pl.BlockSpec((D, _LANES), lambda i: (0, 0)),
            pl.BlockSpec((1, _LANES), lambda i: (0, 0)),
        ],
        out_specs=pl.BlockSpec((BN, C), lambda i: (i, 0)),
        out_shape=jax.ShapeDtypeStruct((N, C), jnp.float32),
        compiler_params=pltpu.CompilerParams(
            dimension_semantics=("parallel",)),
        name="safe_policy_fused",
    )(x, Wp, bp)
